# baseline (speedup 1.0000x reference)
import functools

import jax
import jax.numpy as jnp
from jax import lax
from jax.experimental import pallas as pl
from jax.experimental.pallas import tpu as pltpu
from jax.experimental.pallas import tpu_sc as plsc

NC = 2
NS = 16
CH = 8
NBUF = 4


def kernel(past_k, past_v, past_cu_seqlens, cur_k, cur_v, cur_cu_seqlens):
    nb = past_cu_seqlens.shape[0] - 1
    past_len = past_k.shape[0] // nb
    cur_len = cur_k.shape[0] // nb
    new_len = past_len + cur_len
    tail = past_k.shape[1:]
    total_new = nb * new_len

    workers_per_tensor = NC * NS // 2
    halves = workers_per_tensor // nb
    rows_per_half = past_len // halves
    n_ch = rows_per_half // CH
    n_outer = n_ch // NBUF

    mesh = plsc.VectorSubcoreMesh(core_axis_name="c", subcore_axis_name="s")

    @functools.partial(
        pl.kernel,
        mesh=mesh,
        out_type=[
            jax.ShapeDtypeStruct((total_new,) + tail, past_k.dtype),
            jax.ShapeDtypeStruct((total_new,) + tail, past_v.dtype),
            jax.ShapeDtypeStruct(past_cu_seqlens.shape, past_cu_seqlens.dtype),
        ],
        scratch_types=(
            [pltpu.VMEM((CH,) + tail, past_k.dtype) for _ in range(NBUF)]
            + [pltpu.VMEM((16,), jnp.int32) for _ in range(3)]
            + [pltpu.SemaphoreType.DMA for _ in range(2 * NBUF)]
        ),
    )
    def splice(pk, pv, pcu, ck, cv, ccu, nk, nv, ncu,
               b0, b1, b2, b3, a_v, c_v, o_v,
               g0, g1, g2, g3, s0, s1, s2, s3):
        wid = lax.axis_index("s") * NC + lax.axis_index("c")
        bufs = (b0, b1, b2, b3)
        gs = (g0, g1, g2, g3)
        ss = (s0, s1, s2, s3)

        def stream_tensor(w, past_ref, cur_ref, out_ref):
            b = w // halves
            h = w % halves
            src0 = b * past_len + h * rows_per_half
            dst0 = b * new_len + h * rows_per_half

            def gather(ci, k):
                pltpu.make_async_copy(
                    past_ref.at[pl.ds(src0 + ci * CH, CH)], bufs[k], gs[k]).start()

            def wait_gather(k):
                pltpu.make_async_copy(
                    past_ref.at[pl.ds(src0, CH)], bufs[k], gs[k]).wait()

            def scatter(ci, k):
                pltpu.make_async_copy(
                    bufs[k], out_ref.at[pl.ds(dst0 + ci * CH, CH)], ss[k]).start()

            def wait_scatter(k):
                pltpu.make_async_copy(
                    bufs[k], out_ref.at[pl.ds(dst0, CH)], ss[k]).wait()

            def round_(i, first):
                for k in range(NBUF):
                    if not first:
                        wait_scatter(k)
                    gather(i * NBUF + k, k)
                for k in range(NBUF):
                    wait_gather(k)
                    scatter(i * NBUF + k, k)

            round_(0, True)

            def body(i, carry):
                for k in range(NBUF):
                    wait_scatter(k)
                    pltpu.make_async_copy(
                        past_ref.at[pl.ds(src0 + (i * NBUF + k) * CH, CH)],
                        bufs[k], gs[k]).start()
                for k in range(NBUF):
                    wait_gather(k)
                    pltpu.make_async_copy(
                        bufs[k],
                        out_ref.at[pl.ds(dst0 + (i * NBUF + k) * CH, CH)],
                        ss[k]).start()
                return carry

            lax.fori_loop(1, n_outer, body, 0)

            for k in range(NBUF):
                wait_scatter(k)

            @pl.when(h == halves - 1)
            def _():
                pltpu.sync_copy(cur_ref.at[pl.ds(b * cur_len, cur_len)],
                                bufs[0].at[pl.ds(0, cur_len)])
                pltpu.sync_copy(bufs[0].at[pl.ds(0, cur_len)],
                                out_ref.at[pl.ds(b * new_len + past_len, cur_len)])

        @pl.when(wid < workers_per_tensor)
        def _():
            stream_tensor(wid, pk, ck, nk)

        @pl.when(wid >= workers_per_tensor)
        def _():
            stream_tensor(wid - workers_per_tensor, pv, cv, nv)

        @pl.when(wid == 0)
        def _():
            n = pcu.shape[0]
            pltpu.sync_copy(pcu, a_v.at[pl.ds(0, n)])
            pltpu.sync_copy(ccu, c_v.at[pl.ds(0, n)])
            o_v[...] = a_v[...] + c_v[...]
            pltpu.sync_copy(o_v.at[pl.ds(0, n)], ncu)

    return tuple(splice(past_k, past_v, past_cu_seqlens, cur_k, cur_v, cur_cu_seqlens))
